# Initial kernel scaffold; baseline (speedup 1.0000x reference)
import jax
import jax.numpy as jnp
from jax.experimental import pallas as pl


def kernel(x):
    raise NotImplementedError("write your pallas kernel here")


# speedup vs baseline: 6.8450x; 6.8450x over previous
import functools

import jax
import jax.numpy as jnp
from jax import lax
from jax.experimental import pallas as pl
from jax.experimental.pallas import tpu as pltpu
from jax.experimental.pallas import tpu_sc as plsc

R, N = 128, 32768
K = 256
NC, NS, L = 2, 16, 16
NW = NC * NS
ROWS_PER_W = R // NW
CHUNKS = N // L
HBITS = 11
HBUCKETS = 1 << HBITS


def _scalar(x):
    return x if x.ndim == 0 else x[0]


def _keys(v):
    b = lax.bitcast_convert_type(v, jnp.int32)
    return jnp.where(b >= 0, b, b ^ jnp.int32(0x7FFFFFFF))


def _body(x_hbm, out_hbm, row_v, hist_v, candk_v, candi_v, outrow_v):
    wid = lax.axis_index("s") * NC + lax.axis_index("c")
    iota16 = lax.iota(jnp.int32, L)
    ones = jnp.ones((L,), jnp.int32)

    def do_row(j, _):
        row = wid * ROWS_PER_W + j
        pltpu.sync_copy(x_hbm.at[row], row_v)

        def zero_hist(i, _c):
            hist_v[pl.ds(i * L, L)] = jnp.zeros((L,), jnp.int32)
            return 0

        lax.fori_loop(0, HBUCKETS // L, zero_hist, 0)

        def hist_pass(c, _c):
            key = _keys(row_v[pl.ds(c * L, L)])
            bkt = (key >> (31 - HBITS + 1)) + (HBUCKETS // 2)
            plsc.addupdate_scatter(hist_v, [bkt], ones)
            return 0

        lax.fori_loop(0, CHUNKS, hist_pass, 0)

        def scan_body(i, st):
            acc, bstar, count_above = st
            cb = (HBUCKETS // L - 1) - i
            h = hist_v[pl.ds(cb * L, L)]
            hr = lax.rev(h, (0,))
            cumr = plsc.cumsum(hr)
            tot = cumr[15]
            cross = (acc + cumr) >= K
            take = jnp.logical_and(acc + tot >= K, bstar < 0)
            f = _scalar(plsc.all_reduce_ffs(cross))
            ca_new = acc + jnp.sum(jnp.where(iota16 < f, hr, 0))
            bstar = jnp.where(take, cb * L + (15 - f), bstar)
            count_above = jnp.where(take, ca_new, count_above)
            return (acc + tot, bstar, count_above)

        _, bstar, count_above = lax.fori_loop(
            0, HBUCKETS // L, scan_body,
            (jnp.int32(0), jnp.int32(-1), jnp.int32(0)))

        kp = K - count_above

        def cand_pass(c, off):
            key = _keys(row_v[pl.ds(c * L, L)])
            m = ((key >> (31 - HBITS + 1)) + (HBUCKETS // 2)) == bstar
            cum = plsc.cumsum(m.astype(jnp.int32))
            pos = off + cum - 1
            plsc.store_scatter(candk_v, [pos], key, mask=m)
            plsc.store_scatter(candi_v, [pos], c * L + iota16, mask=m)
            return off + cum[15]

        nc = lax.fori_loop(0, CHUNKS, cand_pass, jnp.int32(0))
        ncc = (nc + (L - 1)) // L

        t0 = (bstar - HBUCKETS // 2) << (31 - HBITS + 1)

        def bit_body(i, t):
            tt = t | (jnp.int32(1) << (31 - HBITS - i))

            def cnt(cc, s):
                key = candk_v[pl.ds(cc * L, L)]
                valid = (cc * L + iota16) < nc
                ge = jnp.logical_and(key >= tt, valid)
                return s + jnp.sum(ge.astype(jnp.int32))

            c_ge = lax.fori_loop(0, ncc, cnt, jnp.int32(0))
            return jnp.where(c_ge >= kp, tt, t)

        tkey = lax.fori_loop(0, 31 - HBITS + 1, bit_body, t0)

        def cnt_gt(cc, s):
            key = candk_v[pl.ds(cc * L, L)]
            valid = (cc * L + iota16) < nc
            gt = jnp.logical_and(key > tkey, valid)
            return s + jnp.sum(gt.astype(jnp.int32))

        c_gt = lax.fori_loop(0, ncc, cnt_gt, jnp.int32(0))
        needed_eq = kp - c_gt

        def eq_scan(cc, st):
            cnt_eq, pos_cut = st
            key = candk_v[pl.ds(cc * L, L)]
            valid = (cc * L + iota16) < nc
            meq = jnp.logical_and(key == tkey, valid)
            cum = plsc.cumsum(meq.astype(jnp.int32))
            hit = jnp.logical_and(meq, (cnt_eq + cum) == needed_eq)
            idxs = candi_v[pl.ds(cc * L, L)]
            got = jnp.sum(jnp.where(hit, idxs, 0))
            anyhit = jnp.sum(hit.astype(jnp.int32)) > 0
            pos_cut = jnp.where(anyhit, got, pos_cut)
            return (cnt_eq + cum[15], pos_cut)

        _, pos_cut = lax.fori_loop(
            0, ncc, eq_scan, (jnp.int32(0), jnp.int32(2**31 - 1)))

        def sel_pass(c, off):
            v = row_v[pl.ds(c * L, L)]
            key = _keys(v)
            idxv = c * L + iota16
            m = jnp.logical_or(
                key > tkey,
                jnp.logical_and(key == tkey, idxv <= pos_cut))
            cum = plsc.cumsum(m.astype(jnp.int32))
            pos = off + cum - 1
            plsc.store_scatter(outrow_v, [pos], v, mask=m)
            return off + cum[15]

        lax.fori_loop(0, CHUNKS, sel_pass, jnp.int32(0))
        pltpu.sync_copy(outrow_v, out_hbm.at[row])
        return 0

    lax.fori_loop(0, ROWS_PER_W, do_row, 0)


_mesh = plsc.VectorSubcoreMesh(
    core_axis_name="c", subcore_axis_name="s", num_cores=NC, num_subcores=NS)

_kmax = pl.kernel(
    _body,
    out_type=jax.ShapeDtypeStruct((R, K), jnp.float32),
    mesh=_mesh,
    scratch_types=[
        pltpu.VMEM((N,), jnp.float32),
        pltpu.VMEM((HBUCKETS,), jnp.int32),
        pltpu.VMEM((N,), jnp.int32),
        pltpu.VMEM((N,), jnp.int32),
        pltpu.VMEM((K,), jnp.float32),
    ],
    compiler_params=pltpu.CompilerParams(needs_layout_passes=False),
)


@jax.jit
def kernel(x):
    return _kmax(x)


# speedup vs baseline: 11.3436x; 1.6572x over previous
import functools

import jax
import jax.numpy as jnp
from jax import lax
from jax.experimental import pallas as pl
from jax.experimental.pallas import tpu as pltpu
from jax.experimental.pallas import tpu_sc as plsc

R, N = 128, 32768
K = 256
NC, NS, L = 2, 16, 16
NW = NC * NS
ROWS_PER_W = R // NW
CHUNKS = N // L
HBITS = 11
HBUCKETS = 1 << HBITS
SHIFT = 32 - HBITS


def _scalar(x):
    return x if x.ndim == 0 else x[0]


def _keys(v):
    b = lax.bitcast_convert_type(v, jnp.int32)
    return jnp.where(b >= 0, b, b ^ jnp.int32(0x7FFFFFFF))


def _vals(k):
    b = jnp.where(k >= 0, k, k ^ jnp.int32(0x7FFFFFFF))
    return lax.bitcast_convert_type(b, jnp.float32)


def _body(x_hbm, out_hbm, row_v, hist_v, candk_v, candi_v, outrow_v):
    wid = lax.axis_index("s") * NC + lax.axis_index("c")
    iota16 = lax.iota(jnp.int32, L)
    ones = jnp.ones((L,), jnp.int32)

    def do_row(j, _):
        row = wid * ROWS_PER_W + j
        pltpu.sync_copy(x_hbm.at[row], row_v)

        def zero_hist(i, _c):
            hist_v[pl.ds(i * L, L)] = jnp.zeros((L,), jnp.int32)
            return 0

        lax.fori_loop(0, HBUCKETS // L, zero_hist, 0)

        def hist_pass(c, _c):
            key = _keys(row_v[pl.ds(c * L, L)])
            bkt = (key >> SHIFT) + (HBUCKETS // 2)
            plsc.addupdate_scatter(hist_v, [bkt], ones)
            return 0

        lax.fori_loop(0, CHUNKS, hist_pass, 0)

        def scan_body(i, st):
            acc, bstar = st
            cb = (HBUCKETS // L - 1) - i
            h = hist_v[pl.ds(cb * L, L)]
            hr = lax.rev(h, (0,))
            cumr = plsc.cumsum(hr)
            tot = cumr[15]
            cross = (acc + cumr) >= K
            take = jnp.logical_and(acc + tot >= K, bstar < 0)
            f = _scalar(plsc.all_reduce_ffs(cross))
            bstar = jnp.where(take, cb * L + (15 - f), bstar)
            return (acc + tot, bstar)

        _, bstar = lax.fori_loop(
            0, HBUCKETS // L, scan_body, (jnp.int32(0), jnp.int32(-1)))

        def cand_pass(c, off):
            key = _keys(row_v[pl.ds(c * L, L)])
            m = ((key >> SHIFT) + (HBUCKETS // 2)) >= bstar
            plsc.store_compressed(candk_v.at[pl.ds(off, L)], key, mask=m)
            plsc.store_compressed(
                candi_v.at[pl.ds(off, L)], c * L + iota16, mask=m)
            return off + _scalar(plsc.all_reduce_population_count(m))

        nc = lax.fori_loop(0, CHUNKS, cand_pass, jnp.int32(0))
        ncc = (nc + (L - 1)) // L

        t0 = (bstar - HBUCKETS // 2) << SHIFT

        def bit_body(i, t):
            tt = t | (jnp.int32(1) << (SHIFT - 1 - i))

            def cnt(cc, s):
                key = candk_v[pl.ds(cc * L, L)]
                valid = (cc * L + iota16) < nc
                ge = jnp.logical_and(key >= tt, valid)
                return s + jnp.sum(ge.astype(jnp.int32))

            c_ge = lax.fori_loop(0, ncc, cnt, jnp.int32(0))
            return jnp.where(c_ge >= K, tt, t)

        tkey = lax.fori_loop(0, SHIFT, bit_body, t0)

        def cnt_gt(cc, s):
            key = candk_v[pl.ds(cc * L, L)]
            valid = (cc * L + iota16) < nc
            gt = jnp.logical_and(key > tkey, valid)
            return s + jnp.sum(gt.astype(jnp.int32))

        c_gt = lax.fori_loop(0, ncc, cnt_gt, jnp.int32(0))
        needed_eq = K - c_gt

        def eq_scan(cc, st):
            cnt_eq, pos_cut = st
            key = candk_v[pl.ds(cc * L, L)]
            valid = (cc * L + iota16) < nc
            meq = jnp.logical_and(key == tkey, valid)
            cum = plsc.cumsum(meq.astype(jnp.int32))
            hit = jnp.logical_and(meq, (cnt_eq + cum) == needed_eq)
            idxs = candi_v[pl.ds(cc * L, L)]
            got = jnp.sum(jnp.where(hit, idxs, 0))
            anyhit = jnp.sum(hit.astype(jnp.int32)) > 0
            pos_cut = jnp.where(anyhit, got, pos_cut)
            return (cnt_eq + cum[15], pos_cut)

        _, pos_cut = lax.fori_loop(
            0, ncc, eq_scan, (jnp.int32(0), jnp.int32(2**31 - 1)))

        def sel_pass(cc, off):
            key = candk_v[pl.ds(cc * L, L)]
            pos = candi_v[pl.ds(cc * L, L)]
            valid = (cc * L + iota16) < nc
            m = jnp.logical_or(
                key > tkey,
                jnp.logical_and(key == tkey, pos <= pos_cut))
            m = jnp.logical_and(m, valid)
            plsc.store_compressed(
                outrow_v.at[pl.ds(off, L)], _vals(key), mask=m)
            return off + _scalar(plsc.all_reduce_population_count(m))

        lax.fori_loop(0, ncc, sel_pass, jnp.int32(0))
        pltpu.sync_copy(outrow_v.at[pl.ds(0, K)], out_hbm.at[row])
        return 0

    lax.fori_loop(0, ROWS_PER_W, do_row, 0)


_mesh = plsc.VectorSubcoreMesh(
    core_axis_name="c", subcore_axis_name="s", num_cores=NC, num_subcores=NS)

_kmax = pl.kernel(
    _body,
    out_type=jax.ShapeDtypeStruct((R, K), jnp.float32),
    mesh=_mesh,
    scratch_types=[
        pltpu.VMEM((N,), jnp.float32),
        pltpu.VMEM((HBUCKETS,), jnp.int32),
        pltpu.VMEM((N,), jnp.int32),
        pltpu.VMEM((N,), jnp.int32),
        pltpu.VMEM((K + L,), jnp.float32),
    ],
    compiler_params=pltpu.CompilerParams(needs_layout_passes=False),
)


@jax.jit
def kernel(x):
    return _kmax(x)


# speedup vs baseline: 36.9563x; 3.2579x over previous
import functools

import jax
import jax.numpy as jnp
from jax import lax
from jax.experimental import pallas as pl
from jax.experimental.pallas import tpu as pltpu
from jax.experimental.pallas import tpu_sc as plsc

R, N = 128, 32768
K = 256
NC, NS, L = 2, 16, 16
NW = NC * NS
ROWS_PER_W = R // NW
CHUNKS = N // L
HBITS = 11
HBUCKETS = 1 << HBITS
SHIFT = 32 - HBITS
T0KEY = 0x40000000


def _scalar(x):
    return x if x.ndim == 0 else x[0]


def _keys(v):
    b = lax.bitcast_convert_type(v, jnp.int32)
    return jnp.where(b >= 0, b, b ^ jnp.int32(0x7FFFFFFF))


def _vals(k):
    b = jnp.where(k >= 0, k, k ^ jnp.int32(0x7FFFFFFF))
    return lax.bitcast_convert_type(b, jnp.float32)


def _body(x_hbm, out_hbm, row_v, hist_v, candk_v, candi_v, outrow_v):
    wid = lax.axis_index("s") * NC + lax.axis_index("c")
    iota16 = lax.iota(jnp.int32, L)
    ones = jnp.ones((L,), jnp.int32)
    zeros16 = jnp.zeros((L,), jnp.int32)

    def count_ge(tt, ncc, nc):
        def cnt(cc, acc):
            key = candk_v[pl.ds(cc * L, L)]
            valid = (cc * L + iota16) < nc
            ge = jnp.logical_and(key >= tt, valid)
            return acc + jnp.where(ge, ones, zeros16)

        return jnp.sum(lax.fori_loop(0, ncc, cnt, zeros16))

    def bit_search(t0, nbits, ncc, nc):
        def bit_body(i, t):
            tt = t | (jnp.int32(1) << (nbits - 1 - i))
            return jnp.where(count_ge(tt, ncc, nc) >= K, tt, t)

        return lax.fori_loop(0, nbits, bit_body, t0)

    def do_row(j, _):
        row = wid * ROWS_PER_W + j
        pltpu.sync_copy(x_hbm.at[row], row_v)

        @plsc.parallel_loop(0, CHUNKS, step=1, unroll=8,
                            carry=jnp.int32(0))
        def nc0(c, off):
            key = _keys(row_v[pl.ds(c * L, L)])
            m = key >= jnp.int32(T0KEY)
            plsc.store_compressed(candk_v.at[pl.ds(off, L)], key, mask=m)
            plsc.store_compressed(
                candi_v.at[pl.ds(off, L)], c * L + iota16, mask=m)
            return off + _scalar(plsc.all_reduce_population_count(m))

        def path_fast(_):
            ncc = (nc0 + (L - 1)) // L
            return bit_search(jnp.int32(T0KEY), 30, ncc, nc0), nc0

        def path_exact(_):
            def zero_hist(i, _c):
                hist_v[pl.ds(i * L, L)] = zeros16
                return 0

            lax.fori_loop(0, HBUCKETS // L, zero_hist, 0)

            def hist_pass(c, _c):
                key = _keys(row_v[pl.ds(c * L, L)])
                bkt = (key >> SHIFT) + (HBUCKETS // 2)
                plsc.addupdate_scatter(hist_v, [bkt], ones)
                return 0

            lax.fori_loop(0, CHUNKS, hist_pass, 0)

            def scan_body(i, st):
                acc, bstar = st
                cb = (HBUCKETS // L - 1) - i
                h = hist_v[pl.ds(cb * L, L)]
                hr = lax.rev(h, (0,))
                cumr = plsc.cumsum(hr)
                tot = cumr[15]
                cross = (acc + cumr) >= K
                take = jnp.logical_and(acc + tot >= K, bstar < 0)
                f = _scalar(plsc.all_reduce_ffs(cross))
                bstar = jnp.where(take, cb * L + (15 - f), bstar)
                return (acc + tot, bstar)

            _, bstar = lax.fori_loop(
                0, HBUCKETS // L, scan_body,
                (jnp.int32(0), jnp.int32(-1)))

            def cand_pass(c, off):
                key = _keys(row_v[pl.ds(c * L, L)])
                m = ((key >> SHIFT) + (HBUCKETS // 2)) >= bstar
                plsc.store_compressed(
                    candk_v.at[pl.ds(off, L)], key, mask=m)
                plsc.store_compressed(
                    candi_v.at[pl.ds(off, L)], c * L + iota16, mask=m)
                return off + _scalar(plsc.all_reduce_population_count(m))

            nc = lax.fori_loop(0, CHUNKS, cand_pass, jnp.int32(0))
            ncc = (nc + (L - 1)) // L
            t0 = (bstar - HBUCKETS // 2) << SHIFT
            return bit_search(t0, SHIFT, ncc, nc), nc

        tkey, nc = lax.cond(nc0 >= K, path_fast, path_exact, 0)
        ncc = (nc + (L - 1)) // L

        def cnt_gt(cc, acc):
            key = candk_v[pl.ds(cc * L, L)]
            valid = (cc * L + iota16) < nc
            gt = jnp.logical_and(key > tkey, valid)
            return acc + jnp.where(gt, ones, zeros16)

        c_gt = jnp.sum(lax.fori_loop(0, ncc, cnt_gt, zeros16))
        needed_eq = K - c_gt

        def eq_scan(cc, st):
            cnt_eq, pos_cut = st
            key = candk_v[pl.ds(cc * L, L)]
            valid = (cc * L + iota16) < nc
            meq = jnp.logical_and(key == tkey, valid)
            cum = plsc.cumsum(meq.astype(jnp.int32))
            hit = jnp.logical_and(meq, (cnt_eq + cum) == needed_eq)
            idxs = candi_v[pl.ds(cc * L, L)]
            got = jnp.sum(jnp.where(hit, idxs, 0))
            anyhit = jnp.sum(hit.astype(jnp.int32)) > 0
            pos_cut = jnp.where(anyhit, got, pos_cut)
            return (cnt_eq + cum[15], pos_cut)

        _, pos_cut = lax.fori_loop(
            0, ncc, eq_scan, (jnp.int32(0), jnp.int32(2**31 - 1)))

        def sel_pass(cc, off):
            key = candk_v[pl.ds(cc * L, L)]
            pos = candi_v[pl.ds(cc * L, L)]
            valid = (cc * L + iota16) < nc
            m = jnp.logical_or(
                key > tkey,
                jnp.logical_and(key == tkey, pos <= pos_cut))
            m = jnp.logical_and(m, valid)
            plsc.store_compressed(
                outrow_v.at[pl.ds(off, L)], _vals(key), mask=m)
            return off + _scalar(plsc.all_reduce_population_count(m))

        lax.fori_loop(0, ncc, sel_pass, jnp.int32(0))
        pltpu.sync_copy(outrow_v.at[pl.ds(0, K)], out_hbm.at[row])
        return 0

    lax.fori_loop(0, ROWS_PER_W, do_row, 0)


_mesh = plsc.VectorSubcoreMesh(
    core_axis_name="c", subcore_axis_name="s", num_cores=NC, num_subcores=NS)

_kmax = pl.kernel(
    _body,
    out_type=jax.ShapeDtypeStruct((R, K), jnp.float32),
    mesh=_mesh,
    scratch_types=[
        pltpu.VMEM((N,), jnp.float32),
        pltpu.VMEM((HBUCKETS,), jnp.int32),
        pltpu.VMEM((N,), jnp.int32),
        pltpu.VMEM((N,), jnp.int32),
        pltpu.VMEM((K + L,), jnp.float32),
    ],
    compiler_params=pltpu.CompilerParams(needs_layout_passes=False),
)


@jax.jit
def kernel(x):
    return _kmax(x)


# speedup vs baseline: 48.8982x; 1.3231x over previous
import functools

import jax
import jax.numpy as jnp
from jax import lax
from jax.experimental import pallas as pl
from jax.experimental.pallas import tpu as pltpu
from jax.experimental.pallas import tpu_sc as plsc

R, N = 128, 32768
K = 256
NC, NS, L = 2, 16, 16
NW = NC * NS
ROWS_PER_W = R // NW
CHUNKS = N // L
HBITS = 11
HBUCKETS = 1 << HBITS
SHIFT = 32 - HBITS
T0KEY = 0x40000000
RBITS = 19
RBUCKETS = 256


def _scalar(x):
    return x if x.ndim == 0 else x[0]


def _keys(v):
    b = lax.bitcast_convert_type(v, jnp.int32)
    return jnp.where(b >= 0, b, b ^ jnp.int32(0x7FFFFFFF))


def _vals(k):
    b = jnp.where(k >= 0, k, k ^ jnp.int32(0x7FFFFFFF))
    return lax.bitcast_convert_type(b, jnp.float32)


def _body(x_hbm, out_hbm, rowa_v, rowb_v, hist_v, candk_v, outrow_v, sem):
    wid = lax.axis_index("s") * NC + lax.axis_index("c")
    iota16 = lax.iota(jnp.int32, L)
    ones = jnp.ones((L,), jnp.int32)
    zeros16 = jnp.zeros((L,), jnp.int32)

    def count_ge_ref(ref, tt, ncc, nvalid):
        def cnt(cc, acc):
            key = _keys(ref[pl.ds(cc * L, L)])
            valid = (cc * L + iota16) < nvalid
            ge = jnp.logical_and(key >= tt, valid)
            return acc + jnp.where(ge, ones, zeros16)

        return jnp.sum(lax.fori_loop(0, ncc, cnt, zeros16))

    def count_ge_cand(tt, ncc, nc):
        def cnt(cc, acc):
            key = candk_v[pl.ds(cc * L, L)]
            valid = (cc * L + iota16) < nc
            ge = jnp.logical_and(key >= tt, valid)
            return acc + jnp.where(ge, ones, zeros16)

        return jnp.sum(lax.fori_loop(0, ncc, cnt, zeros16))

    def bit_search_cand(t0, nbits, ncc, nc):
        def bit_body(i, t):
            tt = t | (jnp.int32(1) << (nbits - 1 - i))
            return jnp.where(count_ge_cand(tt, ncc, nc) >= K, tt, t)

        return lax.fori_loop(0, nbits, bit_body, t0)

    def do_row(row, buf_v):
        @plsc.parallel_loop(0, CHUNKS, step=1, unroll=8,
                            carry=jnp.int32(0))
        def nc0(c, off):
            key = _keys(buf_v[pl.ds(c * L, L)])
            m = key >= jnp.int32(T0KEY)
            plsc.store_compressed(candk_v.at[pl.ds(off, L)], key, mask=m)
            return off + _scalar(plsc.all_reduce_population_count(m))

        def path_fast(_):
            ncc = (nc0 + (L - 1)) // L

            def zero_hist(i, _c):
                hist_v[pl.ds(i * L, L)] = zeros16
                return 0

            lax.fori_loop(0, RBUCKETS // L, zero_hist, 0)

            def hist_pass(cc, _c):
                key = candk_v[pl.ds(cc * L, L)]
                valid = (cc * L + iota16) < nc0
                bkt = jnp.minimum(
                    (key - jnp.int32(T0KEY)) >> RBITS,
                    jnp.int32(RBUCKETS - 1))
                plsc.addupdate_scatter(
                    hist_v, [bkt], jnp.where(valid, ones, zeros16))
                return 0

            lax.fori_loop(0, ncc, hist_pass, 0)

            def scan_body(i, st):
                acc, b8 = st
                cb = (RBUCKETS // L - 1) - i
                h = hist_v[pl.ds(cb * L, L)]
                hr = lax.rev(h, (0,))
                cumr = plsc.cumsum(hr)
                tot = cumr[15]
                cross = (acc + cumr) >= K
                take = jnp.logical_and(acc + tot >= K, b8 < 0)
                f = _scalar(plsc.all_reduce_ffs(cross))
                b8 = jnp.where(take, cb * L + (15 - f), b8)
                return (acc + tot, b8)

            _, b8 = lax.fori_loop(
                0, RBUCKETS // L, scan_body,
                (jnp.int32(0), jnp.int32(-1)))

            def refine_sub(_):
                def sub_pass(cc, off2):
                    key = candk_v[pl.ds(cc * L, L)]
                    valid = (cc * L + iota16) < nc0
                    bkt = jnp.minimum(
                        (key - jnp.int32(T0KEY)) >> RBITS,
                        jnp.int32(RBUCKETS - 1))
                    m = jnp.logical_and(bkt == b8, valid)
                    plsc.store_compressed(
                        buf_v.at[pl.ds(off2, L)], _vals(key), mask=m)
                    return off2 + _scalar(
                        plsc.all_reduce_population_count(m))

                nsub = lax.fori_loop(0, ncc, sub_pass, jnp.int32(0))
                nsc = (nsub + (L - 1)) // L
                t1 = jnp.int32(T0KEY) + (b8 << RBITS)

                def above_cnt(cc, acc):
                    key = candk_v[pl.ds(cc * L, L)]
                    valid = (cc * L + iota16) < nc0
                    bkt = jnp.minimum(
                        (key - jnp.int32(T0KEY)) >> RBITS,
                        jnp.int32(RBUCKETS - 1))
                    gt = jnp.logical_and(bkt > b8, valid)
                    return acc + jnp.where(gt, ones, zeros16)

                n_above = jnp.sum(
                    lax.fori_loop(0, ncc, above_cnt, zeros16))
                ktarget = K - n_above

                def bit_body(i, t):
                    tt = t | (jnp.int32(1) << (RBITS - 1 - i))
                    c_ge = count_ge_ref(buf_v, tt, nsc, nsub)
                    return jnp.where(c_ge >= ktarget, tt, t)

                return lax.fori_loop(0, RBITS, bit_body, t1)

            def refine_full(_):
                return bit_search_cand(jnp.int32(T0KEY), 30, ncc, nc0)

            tkey = lax.cond(
                b8 < jnp.int32(RBUCKETS - 1), refine_sub, refine_full, 0)
            return tkey, nc0

        def path_exact(_):
            def zero_hist(i, _c):
                hist_v[pl.ds(i * L, L)] = zeros16
                return 0

            lax.fori_loop(0, HBUCKETS // L, zero_hist, 0)

            def hist_pass(c, _c):
                key = _keys(buf_v[pl.ds(c * L, L)])
                bkt = (key >> SHIFT) + (HBUCKETS // 2)
                plsc.addupdate_scatter(hist_v, [bkt], ones)
                return 0

            lax.fori_loop(0, CHUNKS, hist_pass, 0)

            def scan_body(i, st):
                acc, bstar = st
                cb = (HBUCKETS // L - 1) - i
                h = hist_v[pl.ds(cb * L, L)]
                hr = lax.rev(h, (0,))
                cumr = plsc.cumsum(hr)
                tot = cumr[15]
                cross = (acc + cumr) >= K
                take = jnp.logical_and(acc + tot >= K, bstar < 0)
                f = _scalar(plsc.all_reduce_ffs(cross))
                bstar = jnp.where(take, cb * L + (15 - f), bstar)
                return (acc + tot, bstar)

            _, bstar = lax.fori_loop(
                0, HBUCKETS // L, scan_body,
                (jnp.int32(0), jnp.int32(-1)))

            def cand_pass(c, off):
                key = _keys(buf_v[pl.ds(c * L, L)])
                m = ((key >> SHIFT) + (HBUCKETS // 2)) >= bstar
                plsc.store_compressed(
                    candk_v.at[pl.ds(off, L)], key, mask=m)
                return off + _scalar(plsc.all_reduce_population_count(m))

            nc = lax.fori_loop(0, CHUNKS, cand_pass, jnp.int32(0))
            ncc = (nc + (L - 1)) // L
            t0 = (bstar - HBUCKETS // 2) << SHIFT
            return bit_search_cand(t0, SHIFT, ncc, nc), nc

        tkey, nc = lax.cond(nc0 >= K, path_fast, path_exact, 0)
        ncc = (nc + (L - 1)) // L

        def cnt_gt(cc, acc):
            key = candk_v[pl.ds(cc * L, L)]
            valid = (cc * L + iota16) < nc
            gt = jnp.logical_and(key > tkey, valid)
            return acc + jnp.where(gt, ones, zeros16)

        c_gt = jnp.sum(lax.fori_loop(0, ncc, cnt_gt, zeros16))
        needed_eq = K - c_gt

        def sel_pass(cc, st):
            off, cnt_eq = st
            key = candk_v[pl.ds(cc * L, L)]
            valid = (cc * L + iota16) < nc
            meq = jnp.logical_and(key == tkey, valid)
            cum = plsc.cumsum(meq.astype(jnp.int32))
            sel_eq = jnp.logical_and(meq, (cnt_eq + cum) <= needed_eq)
            m = jnp.logical_or(
                jnp.logical_and(key > tkey, valid), sel_eq)
            plsc.store_compressed(
                outrow_v.at[pl.ds(off, L)], _vals(key), mask=m)
            return (off + _scalar(plsc.all_reduce_population_count(m)),
                    cnt_eq + cum[15])

        lax.fori_loop(0, ncc, sel_pass, (jnp.int32(0), jnp.int32(0)))
        pltpu.sync_copy(outrow_v.at[pl.ds(0, K)], out_hbm.at[row])

    row0 = wid * ROWS_PER_W
    pltpu.sync_copy(x_hbm.at[row0], rowa_v)
    bufs = [rowa_v, rowb_v]
    handle = None
    for j in range(ROWS_PER_W):
        if j + 1 < ROWS_PER_W:
            handle = pltpu.async_copy(
                x_hbm.at[row0 + j + 1], bufs[(j + 1) % 2], sem)
        do_row(row0 + j, bufs[j % 2])
        if handle is not None:
            handle.wait()
            handle = None


_mesh = plsc.VectorSubcoreMesh(
    core_axis_name="c", subcore_axis_name="s", num_cores=NC, num_subcores=NS)

_kmax = pl.kernel(
    _body,
    out_type=jax.ShapeDtypeStruct((R, K), jnp.float32),
    mesh=_mesh,
    scratch_types=[
        pltpu.VMEM((N,), jnp.float32),
        pltpu.VMEM((N,), jnp.float32),
        pltpu.VMEM((HBUCKETS,), jnp.int32),
        pltpu.VMEM((N,), jnp.int32),
        pltpu.VMEM((K + L,), jnp.float32),
        pltpu.SemaphoreType.DMA,
    ],
    compiler_params=pltpu.CompilerParams(needs_layout_passes=False),
)


@jax.jit
def kernel(x):
    return _kmax(x)


# speedup vs baseline: 52.8490x; 1.0808x over previous
import functools

import jax
import jax.numpy as jnp
from jax import lax
from jax.experimental import pallas as pl
from jax.experimental.pallas import tpu as pltpu
from jax.experimental.pallas import tpu_sc as plsc

R, N = 128, 32768
K = 256
NC, NS, L = 2, 16, 16
NW = NC * NS
ROWS_PER_W = R // NW
CHUNKS = N // L
HBITS = 11
HBUCKETS = 1 << HBITS
SHIFT = 32 - HBITS
T0 = 2.0
T0KEY = 0x40000000
RBITS = 19
RBUCKETS = 256


def _scalar(x):
    return x if x.ndim == 0 else x[0]


def _keys(v):
    b = lax.bitcast_convert_type(v, jnp.int32)
    return jnp.where(b >= 0, b, b ^ jnp.int32(0x7FFFFFFF))


def _body(x_hbm, out_hbm, rowa_v, rowb_v, hist_v, candv_v, outrow_v, sem):
    wid = lax.axis_index("s") * NC + lax.axis_index("c")
    iota16 = lax.iota(jnp.int32, L)
    ones = jnp.ones((L,), jnp.int32)
    zeros16 = jnp.zeros((L,), jnp.int32)

    def count_ge_ref(ref, tt, ncc, nvalid):
        def cnt(cc, acc):
            key = _keys(ref[pl.ds(cc * L, L)])
            valid = (cc * L + iota16) < nvalid
            ge = jnp.logical_and(key >= tt, valid)
            return acc + jnp.where(ge, ones, zeros16)

        return jnp.sum(lax.fori_loop(0, ncc, cnt, zeros16))

    def bit_search(ref, t0, nbits, ncc, nvalid, ktarget):
        def bit_body(i, t):
            tt = t | (jnp.int32(1) << (nbits - 1 - i))
            c_ge = count_ge_ref(ref, tt, ncc, nvalid)
            return jnp.where(c_ge >= ktarget, tt, t)

        return lax.fori_loop(0, nbits, bit_body, t0)

    def do_row(row, buf_v):
        @plsc.parallel_loop(0, CHUNKS, step=1, unroll=8,
                            carry=jnp.int32(0))
        def nc0(c, off):
            v = buf_v[pl.ds(c * L, L)]
            m = v >= jnp.float32(T0)
            plsc.store_compressed(candv_v.at[pl.ds(off, L)], v, mask=m)
            return off + _scalar(plsc.all_reduce_population_count(m))

        def path_fast(_):
            ncc = (nc0 + (L - 1)) // L

            def zero_hist(i, _c):
                hist_v[pl.ds(i * L, L)] = zeros16
                return 0

            lax.fori_loop(0, RBUCKETS // L, zero_hist, 0)

            def hist_pass(cc, _c):
                key = _keys(candv_v[pl.ds(cc * L, L)])
                valid = (cc * L + iota16) < nc0
                bkt = jnp.minimum(
                    (key - jnp.int32(T0KEY)) >> RBITS,
                    jnp.int32(RBUCKETS - 1))
                plsc.addupdate_scatter(
                    hist_v, [bkt], jnp.where(valid, ones, zeros16))
                return 0

            lax.fori_loop(0, ncc, hist_pass, 0)

            def scan_body(i, st):
                acc, b8 = st
                cb = (RBUCKETS // L - 1) - i
                h = hist_v[pl.ds(cb * L, L)]
                hr = lax.rev(h, (0,))
                cumr = plsc.cumsum(hr)
                tot = cumr[15]
                cross = (acc + cumr) >= K
                take = jnp.logical_and(acc + tot >= K, b8 < 0)
                f = _scalar(plsc.all_reduce_ffs(cross))
                b8 = jnp.where(take, cb * L + (15 - f), b8)
                return (acc + tot, b8)

            _, b8 = lax.fori_loop(
                0, RBUCKETS // L, scan_body,
                (jnp.int32(0), jnp.int32(-1)))

            def refine_sub(_):
                def sub_pass(cc, st):
                    off2, nabove = st
                    v = candv_v[pl.ds(cc * L, L)]
                    key = _keys(v)
                    valid = (cc * L + iota16) < nc0
                    bkt = jnp.minimum(
                        (key - jnp.int32(T0KEY)) >> RBITS,
                        jnp.int32(RBUCKETS - 1))
                    m = jnp.logical_and(bkt == b8, valid)
                    gt = jnp.logical_and(bkt > b8, valid)
                    plsc.store_compressed(
                        buf_v.at[pl.ds(off2, L)], v, mask=m)
                    return (
                        off2 + _scalar(
                            plsc.all_reduce_population_count(m)),
                        nabove + _scalar(
                            plsc.all_reduce_population_count(gt)))

                nsub, n_above = lax.fori_loop(
                    0, ncc, sub_pass, (jnp.int32(0), jnp.int32(0)))
                nsc = (nsub + (L - 1)) // L
                t1 = jnp.int32(T0KEY) + (b8 << RBITS)
                ktarget = K - n_above
                tkey = bit_search(buf_v, t1, RBITS, nsc, nsub, ktarget)
                c_gt = n_above + count_ge_ref(
                    buf_v, tkey + 1, nsc, nsub)
                return tkey, c_gt

            def refine_full(_):
                tkey = bit_search(
                    candv_v, jnp.int32(T0KEY), 30, ncc, nc0, K)
                return tkey, count_ge_ref(candv_v, tkey + 1, ncc, nc0)

            tkey, c_gt = lax.cond(
                b8 < jnp.int32(RBUCKETS - 1), refine_sub, refine_full, 0)
            return tkey, nc0, c_gt

        def path_exact(_):
            def zero_hist(i, _c):
                hist_v[pl.ds(i * L, L)] = zeros16
                return 0

            lax.fori_loop(0, HBUCKETS // L, zero_hist, 0)

            def hist_pass(c, _c):
                key = _keys(buf_v[pl.ds(c * L, L)])
                bkt = (key >> SHIFT) + (HBUCKETS // 2)
                plsc.addupdate_scatter(hist_v, [bkt], ones)
                return 0

            lax.fori_loop(0, CHUNKS, hist_pass, 0)

            def scan_body(i, st):
                acc, bstar = st
                cb = (HBUCKETS // L - 1) - i
                h = hist_v[pl.ds(cb * L, L)]
                hr = lax.rev(h, (0,))
                cumr = plsc.cumsum(hr)
                tot = cumr[15]
                cross = (acc + cumr) >= K
                take = jnp.logical_and(acc + tot >= K, bstar < 0)
                f = _scalar(plsc.all_reduce_ffs(cross))
                bstar = jnp.where(take, cb * L + (15 - f), bstar)
                return (acc + tot, bstar)

            _, bstar = lax.fori_loop(
                0, HBUCKETS // L, scan_body,
                (jnp.int32(0), jnp.int32(-1)))

            def cand_pass(c, off):
                v = buf_v[pl.ds(c * L, L)]
                key = _keys(v)
                m = ((key >> SHIFT) + (HBUCKETS // 2)) >= bstar
                plsc.store_compressed(
                    candv_v.at[pl.ds(off, L)], v, mask=m)
                return off + _scalar(plsc.all_reduce_population_count(m))

            nc = lax.fori_loop(0, CHUNKS, cand_pass, jnp.int32(0))
            ncc = (nc + (L - 1)) // L
            t0 = (bstar - HBUCKETS // 2) << SHIFT
            tkey = bit_search(candv_v, t0, SHIFT, ncc, nc, K)
            return tkey, nc, count_ge_ref(candv_v, tkey + 1, ncc, nc)

        tkey, nc, c_gt = lax.cond(nc0 >= K, path_fast, path_exact, 0)
        ncc = (nc + (L - 1)) // L
        needed_eq = K - c_gt

        def sel_pass(cc, st):
            off, cnt_eq = st
            v = candv_v[pl.ds(cc * L, L)]
            key = _keys(v)
            valid = (cc * L + iota16) < nc
            meq = jnp.logical_and(key == tkey, valid)
            cum = plsc.cumsum(meq.astype(jnp.int32))
            sel_eq = jnp.logical_and(meq, (cnt_eq + cum) <= needed_eq)
            m = jnp.logical_or(
                jnp.logical_and(key > tkey, valid), sel_eq)
            plsc.store_compressed(
                outrow_v.at[pl.ds(off, L)], v, mask=m)
            return (off + _scalar(plsc.all_reduce_population_count(m)),
                    cnt_eq + cum[15])

        lax.fori_loop(0, ncc, sel_pass, (jnp.int32(0), jnp.int32(0)))
        pltpu.sync_copy(outrow_v.at[pl.ds(0, K)], out_hbm.at[row])

    row0 = wid * ROWS_PER_W
    pltpu.sync_copy(x_hbm.at[row0], rowa_v)
    bufs = [rowa_v, rowb_v]
    handle = None
    for j in range(ROWS_PER_W):
        if j + 1 < ROWS_PER_W:
            handle = pltpu.async_copy(
                x_hbm.at[row0 + j + 1], bufs[(j + 1) % 2], sem)
        do_row(row0 + j, bufs[j % 2])
        if handle is not None:
            handle.wait()
            handle = None


_mesh = plsc.VectorSubcoreMesh(
    core_axis_name="c", subcore_axis_name="s", num_cores=NC, num_subcores=NS)

_kmax = pl.kernel(
    _body,
    out_type=jax.ShapeDtypeStruct((R, K), jnp.float32),
    mesh=_mesh,
    scratch_types=[
        pltpu.VMEM((N,), jnp.float32),
        pltpu.VMEM((N,), jnp.float32),
        pltpu.VMEM((HBUCKETS,), jnp.int32),
        pltpu.VMEM((N,), jnp.float32),
        pltpu.VMEM((K + L,), jnp.float32),
        pltpu.SemaphoreType.DMA,
    ],
    compiler_params=pltpu.CompilerParams(needs_layout_passes=False),
)


@jax.jit
def kernel(x):
    return _kmax(x)
